# baseline (speedup 1.0000x reference)
import functools

import jax
import jax.numpy as jnp
from jax import lax
from jax.experimental import pallas as pl
from jax.experimental.pallas import tpu as pltpu
from jax.experimental.pallas import tpu_sc as plsc

_NC = 2
_NS = 16
_NW = _NC * _NS


def _make_sc_copy(seq_len, d_model, dtype):
    rows_per_w = seq_len // _NW
    mesh = plsc.VectorSubcoreMesh(core_axis_name="c", subcore_axis_name="s")

    @functools.partial(
        pl.kernel,
        mesh=mesh,
        out_type=jax.ShapeDtypeStruct((seq_len, d_model), dtype),
        scratch_types=[pltpu.SemaphoreType.DMA],
    )
    def sc_copy(table_hbm, out_hbm, sem):
        wid = lax.axis_index("s") * _NC + lax.axis_index("c")
        base = wid * rows_per_w
        pltpu.async_copy(
            table_hbm.at[pl.ds(base, rows_per_w)],
            out_hbm.at[pl.ds(base, rows_per_w)],
            sem,
        ).wait()

    return sc_copy


def kernel(x, table):
    seq_len = x.shape[1]
    d_model = table.shape[1]
    out = _make_sc_copy(seq_len, d_model, table.dtype)(table)
    return out[None, :, :]


# speedup vs baseline: 24.6262x; 24.6262x over previous
import functools

import jax
import jax.numpy as jnp
from jax import lax
from jax.experimental import pallas as pl
from jax.experimental.pallas import tpu as pltpu
from jax.experimental.pallas import tpu_sc as plsc

_NC = 2
_NS = 16
_NW = _NC * _NS
_CHUNK_ROWS = 32


def _make_sc_copy(seq_len, d_model, dtype):
    rows_per_w = seq_len // _NW
    n_chunks = rows_per_w // _CHUNK_ROWS
    mesh = plsc.VectorSubcoreMesh(core_axis_name="c", subcore_axis_name="s")

    @functools.partial(
        pl.kernel,
        mesh=mesh,
        out_type=jax.ShapeDtypeStruct((seq_len, d_model), dtype),
        scratch_types=[
            pltpu.VMEM((2, _CHUNK_ROWS, d_model), dtype),
            pltpu.SemaphoreType.DMA((2,)),
            pltpu.SemaphoreType.DMA((2,)),
        ],
    )
    def sc_copy(table_hbm, out_hbm, buf, in_sems, out_sems):
        wid = lax.axis_index("s") * _NC + lax.axis_index("c")
        base = wid * rows_per_w

        def in_copy(i, b):
            return pltpu.make_async_copy(
                table_hbm.at[pl.ds(base + i * _CHUNK_ROWS, _CHUNK_ROWS)],
                buf.at[b],
                in_sems.at[b],
            )

        def out_copy(i, b):
            return pltpu.make_async_copy(
                buf.at[b],
                out_hbm.at[pl.ds(base + i * _CHUNK_ROWS, _CHUNK_ROWS)],
                out_sems.at[b],
            )

        in_copy(0, 0).start()
        if n_chunks > 1:
            in_copy(1, 1).start()
        for i in range(n_chunks):
            b = i % 2
            in_copy(i, b).wait()
            out_copy(i, b).start()
            if i + 2 < n_chunks:
                out_copy(i, b).wait()
                in_copy(i + 2, b).start()
        for i in range(max(0, n_chunks - 2), n_chunks):
            out_copy(i, i % 2).wait()

    return sc_copy


def kernel(x, table):
    seq_len = x.shape[1]
    d_model = table.shape[1]
    out = _make_sc_copy(seq_len, d_model, table.dtype)(table)
    return out[None, :, :]
